# baseline (speedup 1.0000x reference)
import dataclasses
import functools

import jax
import jax.numpy as jnp
from jax import lax
from jax.experimental import pallas as pl
from jax.experimental.pallas import tpu as pltpu
from jax.experimental.pallas import tpu_sc as plsc

NC = 2
NS = 16
NW = NC * NS
N = 16384
D = 512
C = 1000
CP = 1024
L = 16
MOM = 0.9

NG = 8
NB = 4
CB = D // NB
RPG = N // NG
ZR = 8
NCH = RPG // ZR
YC = 256
RPW = N // NW


def _sc_compiler_params(use_tc_tiling):
    cp = pltpu.CompilerParams()
    fields = pltpu.CompilerParams.__dataclass_fields__
    if "needs_layout_passes" in fields:
        cp = dataclasses.replace(cp, needs_layout_passes=False)
    if not use_tc_tiling and "use_tc_tiling_on_sc" in fields:
        cp = dataclasses.replace(cp, use_tc_tiling_on_sc=False)
    return cp


def _segment_sums_sc(z, y):
    mesh = plsc.VectorSubcoreMesh(core_axis_name="c", subcore_axis_name="s")

    @functools.partial(
        pl.kernel,
        compiler_params=_sc_compiler_params(use_tc_tiling=True),
        out_type=jax.ShapeDtypeStruct((NW * CB * C,), jnp.float32),
        mesh=mesh,
        scratch_types=(
            [pltpu.VMEM((C * L,), jnp.float32)] * (CB // L)
            + [
                pltpu.VMEM((ZR, CB), jnp.float32),
                pltpu.VMEM((ZR, CB), jnp.float32),
                pltpu.VMEM((YC,), jnp.int32),
                pltpu.SemaphoreType.DMA,
                pltpu.SemaphoreType.DMA,
            ]
        ),
    )
    def body(z_hbm, y_hbm, out_hbm, a0, a1, a2, a3, a4, a5, a6, a7,
             z0, z1, y_v, sem0, sem1):
        accs = [a0, a1, a2, a3, a4, a5, a6, a7]
        c = lax.axis_index("c")
        s = lax.axis_index("s")
        wid = s * NC + c
        g = wid // NB
        b = wid % NB
        rowbase = g * RPG
        col0 = b * CB

        zeros16 = jnp.zeros((L,), jnp.float32)
        iota16 = lax.iota(jnp.int32, L)

        @pl.loop(0, C)
        def _(i):
            for acc in accs:
                acc[pl.ds(i * L, L)] = zeros16

        def z_copy(buf, sem, ch):
            return pltpu.make_async_copy(
                z_hbm.at[pl.ds(rowbase + ch * ZR, ZR), pl.ds(col0, CB)],
                buf, sem)

        def process(buf, ch):
            locbase = (ch % (YC // ZR)) * ZR

            def row_idx(i):
                yb = plsc.load_gather(
                    y_v, [jnp.full((L,), locbase + i, jnp.int32)])
                return yb * L + iota16

            def row_vals(i):
                return [buf[i, pl.ds(j * L, L)] for j in range(CB // L)]

            idx, vals = row_idx(0), row_vals(0)
            for i in range(ZR):
                if i + 1 < ZR:
                    nidx, nvals = row_idx(i + 1), row_vals(i + 1)
                else:
                    nidx = nvals = None
                for j in range(CB // L):
                    plsc.addupdate_scatter(accs[j], [idx], vals[j])
                idx, vals = nidx, nvals

        z_copy(z0, sem0, 0).start()
        z_copy(z1, sem1, 1).start()

        @pl.loop(0, NCH // 2)
        def _(h):
            c0 = 2 * h
            c1 = 2 * h + 1

            @pl.when(c0 % (YC // ZR) == 0)
            def _():
                pltpu.sync_copy(
                    y_hbm.at[pl.ds(rowbase + (c0 // (YC // ZR)) * YC, YC)],
                    y_v)

            z_copy(z0, sem0, c0).wait()
            process(z0, c0)

            @pl.when(c0 + 2 < NCH)
            def _():
                z_copy(z0, sem0, c0 + 2).start()

            z_copy(z1, sem1, c1).wait()
            process(z1, c1)

            @pl.when(c1 + 2 < NCH)
            def _():
                z_copy(z1, sem1, c1 + 2).start()

        for j in range(CB // L):
            pltpu.sync_copy(
                accs[j],
                out_hbm.at[pl.ds((wid * (CB // L) + j) * (C * L), C * L)])

    return body(z, y)


def _segment_counts_sc(y):
    mesh = plsc.VectorSubcoreMesh(core_axis_name="c", subcore_axis_name="s")

    @functools.partial(
        pl.kernel,
        compiler_params=_sc_compiler_params(use_tc_tiling=False),
        out_type=jax.ShapeDtypeStruct((NW, 8, CP * L // 8), jnp.float32),
        mesh=mesh,
        scratch_types=[
            pltpu.VMEM((CP, L), jnp.float32),
            pltpu.VMEM((8, CP * L // 8), jnp.float32),
            pltpu.VMEM((RPW,), jnp.int32),
        ],
    )
    def body(y_hbm, cnt_hbm, cnt, stage, y_v):
        c = lax.axis_index("c")
        s = lax.axis_index("s")
        wid = s * NC + c

        zeros16 = jnp.zeros((L,), jnp.float32)
        iota16 = lax.iota(jnp.int32, L)
        ones16 = jnp.ones((L,), jnp.float32)

        @pl.loop(0, CP)
        def _(i):
            cnt[i, :] = zeros16

        pltpu.sync_copy(y_hbm.at[pl.ds(wid * RPW, RPW)], y_v)

        @pl.loop(0, RPW, step=L)
        def _(g):
            y16 = y_v[pl.ds(g, L)]
            plsc.addupdate_scatter(cnt, [y16, iota16], ones16)

        @pl.loop(0, 8)
        def _(j):
            @pl.loop(0, CP // 8)
            def _(k):
                stage[j, pl.ds(k * L, L)] = cnt[j * (CP // 8) + k, :]

        pltpu.sync_copy(stage, cnt_hbm.at[wid])

    return body(y)


def _epilogue_body(sums_ref, cnt_ref, proto_ref, counts_ref,
                   proto_out_ref, counts_out_ref):
    sums = jnp.sum(sums_ref[...], axis=0)

    cnt3 = cnt_ref[...].reshape(NW, CP, L)
    cnt_all = jnp.sum(cnt3, axis=0)
    cnt = jnp.sum(cnt_all, axis=1, keepdims=True)[:C]
    proto = proto_ref[...]

    cnt_safe = jnp.where(cnt > 0, cnt, 1.0)
    z_mean = sums / cnt_safe
    n1 = jnp.sqrt(jnp.sum(z_mean * z_mean, axis=1, keepdims=True))
    z_mean_n = z_mean / jnp.maximum(n1, 1e-12)
    new = MOM * proto + (1.0 - MOM) * z_mean_n
    n2 = jnp.sqrt(jnp.sum(new * new, axis=1, keepdims=True))
    new_n = new / jnp.maximum(n2, 1e-12)
    proto_out_ref[...] = jnp.where(cnt > 0, new_n, proto)
    counts_out_ref[...] = counts_ref[...] + cnt


def kernel(z, y, proto, counts):
    y = y.reshape(-1).astype(jnp.int32)
    sums1d = _segment_sums_sc(z, y)
    sums3 = (sums1d.reshape(NG, NB, CB // L, C, L)
             .transpose(0, 3, 1, 2, 4).reshape(NG, C, D))
    cnt3 = _segment_counts_sc(y)

    proto_new, counts_new = pl.pallas_call(
        _epilogue_body,
        out_shape=[
            jax.ShapeDtypeStruct((C, D), jnp.float32),
            jax.ShapeDtypeStruct((C, 1), jnp.float32),
        ],
    )(sums3, cnt3.reshape(NW, CP * L), proto, counts.reshape(C, 1))

    return proto_new, counts_new.reshape(-1)


# speedup vs baseline: 1.5704x; 1.5704x over previous
import dataclasses
import functools

import jax
import jax.numpy as jnp
from jax import lax
from jax.experimental import pallas as pl
from jax.experimental.pallas import tpu as pltpu
from jax.experimental.pallas import tpu_sc as plsc

NC = 2
NS = 16
NW = NC * NS
N = 16384
D = 512
C = 1000
CP = 1024
L = 16
MOM = 0.9

NG = 8
NB = 4
CB = D // NB
RPG = N // NG
ZR = 8
NCH = RPG // ZR
YC = 256
RPW = N // NW


def _sc_compiler_params(use_tc_tiling):
    cp = pltpu.CompilerParams()
    fields = pltpu.CompilerParams.__dataclass_fields__
    if "needs_layout_passes" in fields:
        cp = dataclasses.replace(cp, needs_layout_passes=False)
    if not use_tc_tiling and "use_tc_tiling_on_sc" in fields:
        cp = dataclasses.replace(cp, use_tc_tiling_on_sc=False)
    return cp


def _segment_sums_sc(z, y):
    mesh = plsc.VectorSubcoreMesh(core_axis_name="c", subcore_axis_name="s")

    @functools.partial(
        pl.kernel,
        compiler_params=_sc_compiler_params(use_tc_tiling=True),
        out_type=jax.ShapeDtypeStruct((NW, C, CB), jnp.float32),
        mesh=mesh,
        scratch_types=(
            [pltpu.VMEM((C * L,), jnp.float32)] * (CB // L)
            + [
                pltpu.VMEM((ZR, CB), jnp.float32),
                pltpu.VMEM((ZR, CB), jnp.float32),
                pltpu.VMEM((YC,), jnp.int32),
                pltpu.SemaphoreType.DMA,
                pltpu.SemaphoreType.DMA,
            ]
        ),
    )
    def body(z_hbm, y_hbm, out_hbm, a0, a1, a2, a3, a4, a5, a6, a7,
             z0, z1, y_v, sem0, sem1):
        accs = [a0, a1, a2, a3, a4, a5, a6, a7]
        c = lax.axis_index("c")
        s = lax.axis_index("s")
        wid = s * NC + c
        g = wid // NB
        b = wid % NB
        rowbase = g * RPG
        col0 = b * CB

        zeros16 = jnp.zeros((L,), jnp.float32)
        iota16 = lax.iota(jnp.int32, L)

        @pl.loop(0, C)
        def _(i):
            for acc in accs:
                acc[pl.ds(i * L, L)] = zeros16

        def z_copy(buf, sem, ch):
            return pltpu.make_async_copy(
                z_hbm.at[pl.ds(rowbase + ch * ZR, ZR), pl.ds(col0, CB)],
                buf, sem)

        def process(buf, ch):
            locbase = (ch % (YC // ZR)) * ZR

            def row_idx(i):
                yb = plsc.load_gather(
                    y_v, [jnp.full((L,), locbase + i, jnp.int32)])
                return yb * L + iota16

            def row_vals(i):
                return [buf[i, pl.ds(j * L, L)] for j in range(CB // L)]

            idx, vals = row_idx(0), row_vals(0)
            for i in range(ZR):
                if i + 1 < ZR:
                    nidx, nvals = row_idx(i + 1), row_vals(i + 1)
                else:
                    nidx = nvals = None
                for j in range(CB // L):
                    plsc.addupdate_scatter(accs[j], [idx], vals[j])
                idx, vals = nidx, nvals

        z_copy(z0, sem0, 0).start()
        z_copy(z1, sem1, 1).start()

        @pl.loop(0, NCH // 2)
        def _(h):
            c0 = 2 * h
            c1 = 2 * h + 1

            @pl.when(c0 % (YC // ZR) == 0)
            def _():
                pltpu.sync_copy(
                    y_hbm.at[pl.ds(rowbase + (c0 // (YC // ZR)) * YC, YC)],
                    y_v)

            z_copy(z0, sem0, c0).wait()
            process(z0, c0)

            @pl.when(c0 + 2 < NCH)
            def _():
                z_copy(z0, sem0, c0 + 2).start()

            z_copy(z1, sem1, c1).wait()
            process(z1, c1)

            @pl.when(c1 + 2 < NCH)
            def _():
                z_copy(z1, sem1, c1 + 2).start()

        def stage_block(buf, p):
            for q in range(8):
                for j in range(CB // L):
                    buf[q, pl.ds(j * L, L)] = accs[j][pl.ds((p * 8 + q) * L, L)]

        def out_copy(buf, sem, p):
            return pltpu.make_async_copy(
                buf, out_hbm.at[wid, pl.ds(p * 8, 8), :], sem)

        nblk = C // 8

        @pl.loop(0, nblk // 2)
        def _(h):
            p0 = 2 * h
            p1 = 2 * h + 1

            @pl.when(h > 0)
            def _():
                out_copy(z0, sem0, p0 - 2).wait()

            stage_block(z0, p0)
            out_copy(z0, sem0, p0).start()

            @pl.when(h > 0)
            def _():
                out_copy(z1, sem1, p1 - 2).wait()

            stage_block(z1, p1)
            out_copy(z1, sem1, p1).start()

        out_copy(z0, sem0, nblk - 3).wait()
        stage_block(z0, nblk - 1)
        out_copy(z0, sem0, nblk - 1).start()
        out_copy(z1, sem1, nblk - 2).wait()
        out_copy(z0, sem0, nblk - 1).wait()

    return body(z, y)


def _segment_counts_sc(y):
    mesh = plsc.VectorSubcoreMesh(core_axis_name="c", subcore_axis_name="s")

    @functools.partial(
        pl.kernel,
        compiler_params=_sc_compiler_params(use_tc_tiling=False),
        out_type=jax.ShapeDtypeStruct((NW, 8, CP * L // 8), jnp.float32),
        mesh=mesh,
        scratch_types=[
            pltpu.VMEM((CP, L), jnp.float32),
            pltpu.VMEM((8, CP * L // 8), jnp.float32),
            pltpu.VMEM((RPW,), jnp.int32),
        ],
    )
    def body(y_hbm, cnt_hbm, cnt, stage, y_v):
        c = lax.axis_index("c")
        s = lax.axis_index("s")
        wid = s * NC + c

        zeros16 = jnp.zeros((L,), jnp.float32)
        iota16 = lax.iota(jnp.int32, L)
        ones16 = jnp.ones((L,), jnp.float32)

        @pl.loop(0, CP)
        def _(i):
            cnt[i, :] = zeros16

        pltpu.sync_copy(y_hbm.at[pl.ds(wid * RPW, RPW)], y_v)

        @pl.loop(0, RPW, step=L)
        def _(g):
            y16 = y_v[pl.ds(g, L)]
            plsc.addupdate_scatter(cnt, [y16, iota16], ones16)

        @pl.loop(0, 8)
        def _(j):
            @pl.loop(0, CP // 8)
            def _(k):
                stage[j, pl.ds(k * L, L)] = cnt[j * (CP // 8) + k, :]

        pltpu.sync_copy(stage, cnt_hbm.at[wid])

    return body(y)


def _epilogue_body(sums_ref, cnt_ref, proto_ref, counts_ref,
                   proto_out_ref, counts_out_ref):
    x = sums_ref[...].reshape(NG, NB, C, CB)
    sblocks = jnp.sum(x, axis=0)
    sums = jnp.concatenate([sblocks[i] for i in range(NB)], axis=1)

    cnt3 = cnt_ref[...].reshape(NW, CP, L)
    cnt_all = jnp.sum(cnt3, axis=0)
    cnt = jnp.sum(cnt_all, axis=1, keepdims=True)[:C]
    proto = proto_ref[...]

    cnt_safe = jnp.where(cnt > 0, cnt, 1.0)
    z_mean = sums / cnt_safe
    n1 = jnp.sqrt(jnp.sum(z_mean * z_mean, axis=1, keepdims=True))
    z_mean_n = z_mean / jnp.maximum(n1, 1e-12)
    new = MOM * proto + (1.0 - MOM) * z_mean_n
    n2 = jnp.sqrt(jnp.sum(new * new, axis=1, keepdims=True))
    new_n = new / jnp.maximum(n2, 1e-12)
    proto_out_ref[...] = jnp.where(cnt > 0, new_n, proto)
    counts_out_ref[...] = counts_ref[...] + cnt


def kernel(z, y, proto, counts):
    y = y.reshape(-1).astype(jnp.int32)
    sums3 = _segment_sums_sc(z, y)
    cnt3 = _segment_counts_sc(y)

    proto_new, counts_new = pl.pallas_call(
        _epilogue_body,
        out_shape=[
            jax.ShapeDtypeStruct((C, D), jnp.float32),
            jax.ShapeDtypeStruct((C, 1), jnp.float32),
        ],
    )(sums3, cnt3.reshape(NW, CP * L), proto, counts.reshape(C, 1))

    return proto_new, counts_new.reshape(-1)


# speedup vs baseline: 1.6142x; 1.0279x over previous
import dataclasses
import functools

import jax
import jax.numpy as jnp
from jax import lax
from jax.experimental import pallas as pl
from jax.experimental.pallas import tpu as pltpu
from jax.experimental.pallas import tpu_sc as plsc

NC = 2
NS = 16
NW = NC * NS
N = 16384
D = 512
C = 1000
CP = 1024
L = 16
MOM = 0.9

NG = 8
NB = 4
CB = D // NB
RPG = N // NG
ZR = 8
BR = 32
NBIG = RPG // BR
YC = 256
RPW = N // NW


def _sc_compiler_params(use_tc_tiling):
    cp = pltpu.CompilerParams()
    fields = pltpu.CompilerParams.__dataclass_fields__
    if "needs_layout_passes" in fields:
        cp = dataclasses.replace(cp, needs_layout_passes=False)
    if not use_tc_tiling and "use_tc_tiling_on_sc" in fields:
        cp = dataclasses.replace(cp, use_tc_tiling_on_sc=False)
    return cp


def _segment_sums_sc(z, y):
    mesh = plsc.VectorSubcoreMesh(core_axis_name="c", subcore_axis_name="s")

    @functools.partial(
        pl.kernel,
        compiler_params=_sc_compiler_params(use_tc_tiling=True),
        out_type=[
            jax.ShapeDtypeStruct((NW, C, CB), jnp.float32),
            jax.ShapeDtypeStruct((NW * C * L,), jnp.float32),
        ],
        mesh=mesh,
        scratch_types=(
            [pltpu.VMEM((C * L,), jnp.float32)] * (CB // L)
            + [
                pltpu.VMEM((ZR, CB), jnp.float32),
                pltpu.VMEM((ZR, CB), jnp.float32),
                pltpu.VMEM((YC,), jnp.int32),
                pltpu.SemaphoreType.DMA,
                pltpu.SemaphoreType.DMA,
            ]
        ),
    )
    def body(z_hbm, y_hbm, out_hbm, cnt_hbm, a0, a1, a2, a3, a4, a5, a6, a7,
             z0, z1, y_v, sem0, sem1):
        accs = [a0, a1, a2, a3, a4, a5, a6, a7]
        c = lax.axis_index("c")
        s = lax.axis_index("s")
        wid = s * NC + c
        g = wid // NB
        b = wid % NB
        rowbase = g * RPG
        col0 = b * CB

        zeros16 = jnp.zeros((L,), jnp.float32)
        ones16 = jnp.ones((L,), jnp.float32)
        iota16 = lax.iota(jnp.int32, L)

        def z_copy(buf, sem, ch):
            return pltpu.make_async_copy(
                z_hbm.at[pl.ds(rowbase + ch * ZR, ZR), pl.ds(col0, CB)],
                buf, sem)

        z_copy(z0, sem0, 0).start()
        z_copy(z1, sem1, 1).start()
        pltpu.sync_copy(y_hbm.at[pl.ds(rowbase, YC)], y_v)

        @pl.loop(0, C)
        def _(i):
            for acc in accs:
                acc[pl.ds(i * L, L)] = zeros16

        def process(buf, loc0):
            def row_idx(i):
                yb = plsc.load_gather(
                    y_v, [jnp.full((L,), loc0 + i, jnp.int32)])
                return yb * L + iota16

            def row_vals(i):
                return [buf[i, pl.ds(j * L, L)] for j in range(CB // L)]

            idx, vals = row_idx(0), row_vals(0)
            for i in range(ZR):
                if i + 1 < ZR:
                    nidx, nvals = row_idx(i + 1), row_vals(i + 1)
                else:
                    nidx = nvals = None
                for j in range(CB // L):
                    plsc.addupdate_scatter(accs[j], [idx], vals[j])
                idx, vals = nidx, nvals

        NCH = RPG // ZR

        @pl.loop(0, NCH // 2)
        def _(h):
            c0 = 2 * h
            c1 = 2 * h + 1

            @pl.when((c0 % (YC // ZR) == 0) & (c0 > 0))
            def _():
                pltpu.sync_copy(
                    y_hbm.at[pl.ds(rowbase + (c0 // (YC // ZR)) * YC, YC)],
                    y_v)

            locbase = (c0 % (YC // ZR)) * ZR
            z_copy(z0, sem0, c0).wait()
            process(z0, locbase)

            @pl.when(c0 + 2 < NCH)
            def _():
                z_copy(z0, sem0, c0 + 2).start()

            z_copy(z1, sem1, c1).wait()
            process(z1, locbase + ZR)

            @pl.when(c1 + 2 < NCH)
            def _():
                z_copy(z1, sem1, c1 + 2).start()

        def stage_block(buf, p):
            for q in range(8):
                for j in range(CB // L):
                    buf[q, pl.ds(j * L, L)] = accs[j][pl.ds((p * 8 + q) * L, L)]

        def out_copy(buf, sem, p):
            return pltpu.make_async_copy(
                buf, out_hbm.at[wid, pl.ds(p * 8, 8), :], sem)

        nblk = C // 8

        @pl.loop(0, nblk // 2)
        def _(h):
            p0 = 2 * h
            p1 = 2 * h + 1

            @pl.when(h > 0)
            def _():
                out_copy(z0, sem0, p0 - 2).wait()

            stage_block(z0, p0)
            out_copy(z0, sem0, p0).start()

            @pl.when(h > 0)
            def _():
                out_copy(z1, sem1, p1 - 2).wait()

            stage_block(z1, p1)
            out_copy(z1, sem1, p1).start()

        out_copy(z0, sem0, nblk - 3).wait()
        stage_block(z0, nblk - 1)
        out_copy(z0, sem0, nblk - 1).start()
        out_copy(z1, sem1, nblk - 2).wait()
        out_copy(z0, sem0, nblk - 1).wait()

        @pl.loop(0, C)
        def _(i):
            a0[pl.ds(i * L, L)] = zeros16

        for t in range(RPW // YC):
            pltpu.sync_copy(
                y_hbm.at[pl.ds(wid * RPW + t * YC, YC)], y_v)

            @pl.loop(0, YC, step=L)
            def _(q):
                y16 = y_v[pl.ds(q, L)]
                plsc.addupdate_scatter(a0, [y16 * L + iota16], ones16)

        pltpu.sync_copy(a0, cnt_hbm.at[pl.ds(wid * C * L, C * L)])

    return body(z, y)


def _epilogue_body(sums_ref, cnt_ref, proto_ref, counts_ref,
                   proto_out_ref, counts_out_ref):
    x = sums_ref[...].reshape(NG, NB, C, CB)
    sblocks = jnp.sum(x, axis=0)
    sums = jnp.concatenate([sblocks[i] for i in range(NB)], axis=1)

    cnt3 = cnt_ref[...].reshape(NW, C, L)
    cnt_all = jnp.sum(cnt3, axis=0)
    cnt = jnp.sum(cnt_all, axis=1, keepdims=True)
    proto = proto_ref[...]

    cnt_safe = jnp.where(cnt > 0, cnt, 1.0)
    z_mean = sums / cnt_safe
    n1 = jnp.sqrt(jnp.sum(z_mean * z_mean, axis=1, keepdims=True))
    z_mean_n = z_mean / jnp.maximum(n1, 1e-12)
    new = MOM * proto + (1.0 - MOM) * z_mean_n
    n2 = jnp.sqrt(jnp.sum(new * new, axis=1, keepdims=True))
    new_n = new / jnp.maximum(n2, 1e-12)
    proto_out_ref[...] = jnp.where(cnt > 0, new_n, proto)
    counts_out_ref[...] = counts_ref[...] + cnt


def kernel(z, y, proto, counts):
    y = y.reshape(-1).astype(jnp.int32)
    sums3, cnt1d = _segment_sums_sc(z, y)

    proto_new, counts_new = pl.pallas_call(
        _epilogue_body,
        out_shape=[
            jax.ShapeDtypeStruct((C, D), jnp.float32),
            jax.ShapeDtypeStruct((C, 1), jnp.float32),
        ],
    )(sums3, cnt1d.reshape(NW, C * L), proto, counts.reshape(C, 1))

    return proto_new, counts_new.reshape(-1)


# speedup vs baseline: 1.8198x; 1.1274x over previous
import dataclasses
import functools

import jax
import jax.numpy as jnp
from jax import lax
from jax.experimental import pallas as pl
from jax.experimental.pallas import tpu as pltpu
from jax.experimental.pallas import tpu_sc as plsc

NC = 2
NS = 16
NW = NC * NS
N = 16384
D = 512
C = 1000
L = 16
MOM = 0.9

NG = 8
NB = 4
CB = D // NB
PC = CB // 2
JP = PC // L
RPG = N // NG
ZR = 8
NCH = RPG // ZR
RZ = 8
YC = 256
RPW = N // NW
NBLK = C // 8
SR = 4


def _sc_compiler_params():
    cp = pltpu.CompilerParams()
    fields = pltpu.CompilerParams.__dataclass_fields__
    if "needs_layout_passes" in fields:
        cp = dataclasses.replace(cp, needs_layout_passes=False)
    return cp


def _segment_sums_sc(z, y):
    mesh = plsc.VectorSubcoreMesh(core_axis_name="c", subcore_axis_name="s")

    @functools.partial(
        pl.kernel,
        compiler_params=_sc_compiler_params(),
        out_type=[
            jax.ShapeDtypeStruct((NW, C, PC), jnp.float32),
            jax.ShapeDtypeStruct((NW, C, PC), jnp.float32),
            jax.ShapeDtypeStruct((NW * C * L,), jnp.float32),
        ],
        mesh=mesh,
        scratch_types=(
            [pltpu.VMEM((C * L,), jnp.float32)] * JP
            + [pltpu.VMEM((ZR, CB), jnp.float32)] * RZ
            + [pltpu.VMEM((8, PC), jnp.float32)] * SR
            + [pltpu.VMEM((YC,), jnp.int32)]
            + [pltpu.SemaphoreType.DMA] * (RZ + SR)
        ),
    )
    def body(z_hbm, y_hbm, outa_hbm, outb_hbm, cnt_hbm, *refs):
        accs = refs[:JP]
        zb = refs[JP:JP + RZ]
        sb = refs[JP + RZ:JP + RZ + SR]
        y_v = refs[JP + RZ + SR]
        zs = refs[JP + RZ + SR + 1:JP + RZ + SR + 1 + RZ]
        ss = refs[JP + RZ + SR + 1 + RZ:]

        c = lax.axis_index("c")
        s = lax.axis_index("s")
        wid = s * NC + c
        g = wid // NB
        b = wid % NB
        rowbase = g * RPG
        col0 = b * CB

        zeros16 = jnp.zeros((L,), jnp.float32)
        ones16 = jnp.ones((L,), jnp.float32)
        iota16 = lax.iota(jnp.int32, L)

        def z_copy(r, ch):
            return pltpu.make_async_copy(
                z_hbm.at[pl.ds(rowbase + ch * ZR, ZR), pl.ds(col0, CB)],
                zb[r], zs[r])

        def process(buf, loc0, p):
            def row_idx(i):
                yb = plsc.load_gather(
                    y_v, [jnp.full((L,), loc0 + i, jnp.int32)])
                return yb * L + iota16

            def row_vals(i):
                return [buf[i, pl.ds(p * PC + j * L, L)] for j in range(JP)]

            idx, vals = row_idx(0), row_vals(0)
            for i in range(ZR):
                if i + 1 < ZR:
                    nidx, nvals = row_idx(i + 1), row_vals(i + 1)
                else:
                    nidx = nvals = None
                for j in range(JP):
                    plsc.addupdate_scatter(accs[j], [idx], vals[j])
                idx, vals = nidx, nvals

        def stage4(buf, pblk):
            for q in range(8):
                for j in range(JP):
                    buf[q, pl.ds(j * L, L)] = \
                        accs[j][pl.ds((pblk * 8 + q) * L, L)]

        for p, out_hbm in enumerate((outa_hbm, outb_hbm)):
            @pl.loop(0, C)
            def _(i):
                for acc in accs:
                    acc[pl.ds(i * L, L)] = zeros16

            for r in range(RZ):
                z_copy(r, r).start()

            @pl.loop(0, NCH // RZ)
            def _(h):
                @pl.when(h % (YC // (ZR * RZ)) == 0)
                def _():
                    pltpu.sync_copy(
                        y_hbm.at[pl.ds(
                            rowbase + (h // (YC // (ZR * RZ))) * YC, YC)],
                        y_v)

                for r in range(RZ):
                    ch = h * RZ + r
                    z_copy(r, ch).wait()
                    process(zb[r], ((h % 4) * RZ + r) * ZR, p)

                    @pl.when(h + 1 < NCH // RZ)
                    def _():
                        z_copy(r, ch + RZ).start()

            def out_copy(r, pblk):
                return pltpu.make_async_copy(
                    sb[r], out_hbm.at[wid, pl.ds(pblk * 8, 8), :], ss[r])

            @pl.loop(0, NBLK // SR)
            def _(hh):
                for r in range(SR):
                    pblk = hh * SR + r

                    @pl.when(hh > 0)
                    def _():
                        out_copy(r, pblk - SR).wait()

                    stage4(sb[r], pblk)
                    out_copy(r, pblk).start()

            out_copy(0, NBLK - 5).wait()
            stage4(sb[0], NBLK - 1)
            out_copy(0, NBLK - 1).start()
            for r in range(1, SR):
                out_copy(r, NBLK - 5 + r).wait()
            out_copy(0, NBLK - 1).wait()

        @pl.loop(0, C)
        def _(i):
            accs[0][pl.ds(i * L, L)] = zeros16

        for t in range(RPW // YC):
            pltpu.sync_copy(
                y_hbm.at[pl.ds(wid * RPW + t * YC, YC)], y_v)

            @pl.loop(0, YC, step=L)
            def _(q):
                y16 = y_v[pl.ds(q, L)]
                plsc.addupdate_scatter(accs[0], [y16 * L + iota16], ones16)

        pltpu.sync_copy(accs[0], cnt_hbm.at[pl.ds(wid * C * L, C * L)])

    return body(z, y)


def _epilogue_body(sa_ref, sb_ref, cnt_ref, proto_ref, counts_ref,
                   proto_out_ref, counts_out_ref):
    xa = jnp.sum(sa_ref[...].reshape(NG, NB, C, PC), axis=0)
    xb = jnp.sum(sb_ref[...].reshape(NG, NB, C, PC), axis=0)
    pieces = []
    for i in range(NB):
        pieces.append(xa[i])
        pieces.append(xb[i])
    sums = jnp.concatenate(pieces, axis=1)

    cnt3 = cnt_ref[...].reshape(NW, C, L)
    cnt_all = jnp.sum(cnt3, axis=0)
    cnt = jnp.sum(cnt_all, axis=1, keepdims=True)
    proto = proto_ref[...]

    cnt_safe = jnp.where(cnt > 0, cnt, 1.0)
    z_mean = sums / cnt_safe
    n1 = jnp.sqrt(jnp.sum(z_mean * z_mean, axis=1, keepdims=True))
    z_mean_n = z_mean / jnp.maximum(n1, 1e-12)
    new = MOM * proto + (1.0 - MOM) * z_mean_n
    n2 = jnp.sqrt(jnp.sum(new * new, axis=1, keepdims=True))
    new_n = new / jnp.maximum(n2, 1e-12)
    proto_out_ref[...] = jnp.where(cnt > 0, new_n, proto)
    counts_out_ref[...] = counts_ref[...] + cnt


def kernel(z, y, proto, counts):
    y = y.reshape(-1).astype(jnp.int32)
    sums_a, sums_b, cnt1d = _segment_sums_sc(z, y)

    proto_new, counts_new = pl.pallas_call(
        _epilogue_body,
        out_shape=[
            jax.ShapeDtypeStruct((C, D), jnp.float32),
            jax.ShapeDtypeStruct((C, 1), jnp.float32),
        ],
    )(sums_a, sums_b, cnt1d.reshape(NW, C * L), proto, counts.reshape(C, 1))

    return proto_new, counts_new.reshape(-1)


# speedup vs baseline: 1.9248x; 1.0577x over previous
import dataclasses
import functools

import jax
import jax.numpy as jnp
from jax import lax
from jax.experimental import pallas as pl
from jax.experimental.pallas import tpu as pltpu
from jax.experimental.pallas import tpu_sc as plsc

NC = 2
NS = 16
NW = NC * NS
N = 16384
D = 512
C = 1000
L = 16
MOM = 0.9

NG = 8
NB = 4
CB = D // NB
PC = CB // 2
JP = PC // L
RPG = N // NG
ZR = 8
NCH = RPG // ZR
RZ = 8
YC = 256
RPW = N // NW
NBLK = C // 8
SR = 4


def _sc_compiler_params():
    cp = pltpu.CompilerParams()
    fields = pltpu.CompilerParams.__dataclass_fields__
    if "needs_layout_passes" in fields:
        cp = dataclasses.replace(cp, needs_layout_passes=False)
    return cp


def _segment_sums_sc(z, y):
    mesh = plsc.VectorSubcoreMesh(core_axis_name="c", subcore_axis_name="s")

    @functools.partial(
        pl.kernel,
        compiler_params=_sc_compiler_params(),
        out_type=[
            jax.ShapeDtypeStruct((NW, C, PC), jnp.float32),
            jax.ShapeDtypeStruct((NW, C, PC), jnp.float32),
            jax.ShapeDtypeStruct((NW * C * L,), jnp.float32),
        ],
        mesh=mesh,
        scratch_types=(
            [pltpu.VMEM((C * L,), jnp.float32)] * JP
            + [pltpu.VMEM((ZR, CB), jnp.float32)] * RZ
            + [pltpu.VMEM((8, PC), jnp.float32)] * SR
            + [pltpu.VMEM((YC,), jnp.int32)]
            + [pltpu.SemaphoreType.DMA] * (RZ + SR)
        ),
    )
    def body(z_hbm, y_hbm, outa_hbm, outb_hbm, cnt_hbm, *refs):
        accs = refs[:JP]
        zb = refs[JP:JP + RZ]
        sb = refs[JP + RZ:JP + RZ + SR]
        y_v = refs[JP + RZ + SR]
        zs = refs[JP + RZ + SR + 1:JP + RZ + SR + 1 + RZ]
        ss = refs[JP + RZ + SR + 1 + RZ:]

        c = lax.axis_index("c")
        s = lax.axis_index("s")
        wid = s * NC + c
        g = wid // NB
        b = wid % NB
        rowbase = g * RPG
        col0 = b * CB

        zeros16 = jnp.zeros((L,), jnp.float32)
        ones16 = jnp.ones((L,), jnp.float32)
        iota16 = lax.iota(jnp.int32, L)

        def z_copy(r, ch):
            return pltpu.make_async_copy(
                z_hbm.at[pl.ds(rowbase + ch * ZR, ZR), pl.ds(col0, CB)],
                zb[r], zs[r])

        def process(buf, loc0, p):
            def row_idx(i):
                yb = plsc.load_gather(
                    y_v, [jnp.full((L,), loc0 + i, jnp.int32)])
                return yb * L + iota16

            def row_vals(i):
                return [buf[i, pl.ds(p * PC + j * L, L)] for j in range(JP)]

            idx, vals = row_idx(0), row_vals(0)
            for i in range(ZR):
                if i + 1 < ZR:
                    nidx, nvals = row_idx(i + 1), row_vals(i + 1)
                else:
                    nidx = nvals = None
                for j in range(JP):
                    plsc.addupdate_scatter(accs[j], [idx], vals[j])
                idx, vals = nidx, nvals

        def stage4(buf, pblk):
            for q in range(8):
                for j in range(JP):
                    buf[q, pl.ds(j * L, L)] = \
                        accs[j][pl.ds((pblk * 8 + q) * L, L)]

        for p, out_hbm in enumerate((outa_hbm, outb_hbm)):
            for r in range(RZ):
                z_copy(r, r).start()

            @pl.loop(0, C, step=8)
            def _(i):
                for u in range(8):
                    for acc in accs:
                        acc[pl.ds((i + u) * L, L)] = zeros16

            @pl.loop(0, NCH // RZ)
            def _(h):
                @pl.when(h % (YC // (ZR * RZ)) == 0)
                def _():
                    pltpu.sync_copy(
                        y_hbm.at[pl.ds(
                            rowbase + (h // (YC // (ZR * RZ))) * YC, YC)],
                        y_v)

                for r in range(RZ):
                    ch = h * RZ + r
                    z_copy(r, ch).wait()
                    process(zb[r], ((h % 4) * RZ + r) * ZR, p)

                    @pl.when(h + 1 < NCH // RZ)
                    def _():
                        z_copy(r, ch + RZ).start()

            def out_copy(r, pblk):
                return pltpu.make_async_copy(
                    sb[r], out_hbm.at[wid, pl.ds(pblk * 8, 8), :], ss[r])

            @pl.loop(0, NBLK // SR)
            def _(hh):
                for r in range(SR):
                    pblk = hh * SR + r

                    @pl.when(hh > 0)
                    def _():
                        out_copy(r, pblk - SR).wait()

                    stage4(sb[r], pblk)
                    out_copy(r, pblk).start()

            out_copy(0, NBLK - 5).wait()
            stage4(sb[0], NBLK - 1)
            out_copy(0, NBLK - 1).start()
            for r in range(1, SR):
                out_copy(r, NBLK - 5 + r).wait()
            out_copy(0, NBLK - 1).wait()

        @pl.loop(0, C, step=8)
        def _(i):
            for u in range(8):
                accs[0][pl.ds((i + u) * L, L)] = zeros16

        for t in range(RPW // YC):
            pltpu.sync_copy(
                y_hbm.at[pl.ds(wid * RPW + t * YC, YC)], y_v)

            @pl.loop(0, YC, step=L)
            def _(q):
                y16 = y_v[pl.ds(q, L)]
                plsc.addupdate_scatter(accs[0], [y16 * L + iota16], ones16)

        pltpu.sync_copy(accs[0], cnt_hbm.at[pl.ds(wid * C * L, C * L)])

    return body(z, y)


def _epilogue_body(sa_ref, sb_ref, cnt_ref, proto_ref, counts_ref,
                   proto_out_ref, counts_out_ref):
    xa = jnp.sum(sa_ref[...].reshape(NG, NB, C, PC), axis=0)
    xb = jnp.sum(sb_ref[...].reshape(NG, NB, C, PC), axis=0)
    pieces = []
    for i in range(NB):
        pieces.append(xa[i])
        pieces.append(xb[i])
    sums = jnp.concatenate(pieces, axis=1)

    cnt3 = cnt_ref[...].reshape(NW, C, L)
    cnt_all = jnp.sum(cnt3, axis=0)
    cnt = jnp.sum(cnt_all, axis=1, keepdims=True)
    proto = proto_ref[...]

    cnt_safe = jnp.where(cnt > 0, cnt, 1.0)
    z_mean = sums / cnt_safe
    n1 = jnp.sqrt(jnp.sum(z_mean * z_mean, axis=1, keepdims=True))
    z_mean_n = z_mean / jnp.maximum(n1, 1e-12)
    new = MOM * proto + (1.0 - MOM) * z_mean_n
    n2 = jnp.sqrt(jnp.sum(new * new, axis=1, keepdims=True))
    new_n = new / jnp.maximum(n2, 1e-12)
    proto_out_ref[...] = jnp.where(cnt > 0, new_n, proto)
    counts_out_ref[...] = counts_ref[...] + cnt


def kernel(z, y, proto, counts):
    y = y.reshape(-1).astype(jnp.int32)
    sums_a, sums_b, cnt1d = _segment_sums_sc(z, y)

    proto_new, counts_new = pl.pallas_call(
        _epilogue_body,
        out_shape=[
            jax.ShapeDtypeStruct((C, D), jnp.float32),
            jax.ShapeDtypeStruct((C, 1), jnp.float32),
        ],
    )(sums_a, sums_b, cnt1d.reshape(NW, C * L), proto, counts.reshape(C, 1))

    return proto_new, counts_new.reshape(-1)
